# baseline (speedup 1.0000x reference)
import functools

import jax
import jax.numpy as jnp
from jax import lax
from jax.experimental import pallas as pl
from jax.experimental.pallas import tpu as pltpu
from jax.experimental.pallas import tpu_sc as plsc

N = 10000
E = 320000
D = 128
H = 16
C = 10

_NW = 32
_EPW = E // _NW
_B = 128
_NBUF = 4
_K = -(-(-(-_EPW // _B)) // _NBUF) * _NBUF
_EPW_PAD = _K * _B
_ACC_ROWS = 10112
_ZROWS = _ACC_ROWS // 16
_OROWS = _ACC_ROWS // 16


def _seg_body(table_hbm, src_hbm, dst_hbm, zeros_hbm, out_hbm,
              src_v, dst_v, rows_v, acc_sh, *sems):
    gsems = sems[:_NBUF]
    ssems = sems[_NBUF:]
    c = lax.axis_index("c")
    s = lax.axis_index("s")
    wid = c * 16 + s
    pltpu.sync_copy(src_hbm.at[wid], src_v)
    pltpu.sync_copy(dst_hbm.at[wid], dst_v)
    pltpu.sync_copy(zeros_hbm, acc_sh.at[pl.ds(s * _ZROWS, _ZROWS)])
    plsc.subcore_barrier()

    def _gather(i, b):
        return pltpu.make_async_copy(
            table_hbm.at[src_v.at[i]], rows_v.at[b], gsems[b])

    def _scatter(i, b):
        return pltpu.make_async_copy(
            rows_v.at[b], acc_sh.at[dst_v.at[i]], ssems[b])

    _gather(0, 0).start()

    @pl.loop(0, _K, step=_NBUF)
    def group(j):
        for b in range(_NBUF):
            _gather(j + b, b).wait()
            nxt = j + b + 1
            nb = (b + 1) % _NBUF
            if b + 1 < _NBUF:
                @pl.when(nxt >= _NBUF)
                def _():
                    _scatter(nxt - _NBUF, nb).wait()
                _gather(nxt, nb).start()
            else:
                @pl.when(nxt < _K)
                def _():
                    _scatter(nxt - _NBUF, nb).wait()
                    _gather(nxt, nb).start()
            _scatter(j + b, b).start(add=True)

    for b in range(_NBUF):
        _scatter(_K - _NBUF + b, b).wait()
    plsc.subcore_barrier()
    pltpu.sync_copy(acc_sh.at[pl.ds(s * _OROWS, _OROWS)],
                    out_hbm.at[c, pl.ds(s * _OROWS, _OROWS)])


_seg_partial = functools.partial(
    pl.kernel,
    out_type=jax.ShapeDtypeStruct((2, _ACC_ROWS, H), jnp.float32),
    mesh=plsc.VectorSubcoreMesh(core_axis_name="c", subcore_axis_name="s"),
    compiler_params=pltpu.CompilerParams(use_tc_tiling_on_sc=False),
    scratch_types=[
        pltpu.VMEM((_K, _B), jnp.int32),
        pltpu.VMEM((_K, _B), jnp.int32),
        pltpu.VMEM((_NBUF, _B, H), jnp.float32),
        pltpu.VMEM_SHARED((_ACC_ROWS, H), jnp.float32),
    ] + [pltpu.SemaphoreType.DMA] * (2 * _NBUF),
)(_seg_body)


_BLK = 2000


def _proj_body(x_ref, w_ref, b_ref, o_ref):
    o_ref[...] = lax.dot_general(
        x_ref[...], w_ref[...], (((1,), (1,)), ((), ())),
        preferred_element_type=jnp.float32) + b_ref[...]


def _proj(xx, wcat, bcat):
    n, d = xx.shape
    m = wcat.shape[0]
    return pl.pallas_call(
        _proj_body,
        grid=(n // _BLK,),
        in_specs=[
            pl.BlockSpec((_BLK, d), lambda i: (i, 0)),
            pl.BlockSpec((m, d), lambda i: (0, 0)),
            pl.BlockSpec((1, m), lambda i: (0, 0)),
        ],
        out_specs=pl.BlockSpec((_BLK, m), lambda i: (i, 0)),
        out_shape=jax.ShapeDtypeStruct((n, m), jnp.float32),
    )(xx, wcat, bcat)


def _mid_body(pa_ref, r_ref, w_ref, b_ref, o_ref):
    agg = pa_ref[0] + pa_ref[1]
    h = jnp.maximum(agg + r_ref[...], 0.0)
    o_ref[...] = lax.dot_general(
        h, w_ref[...], (((1,), (1,)), ((), ())),
        preferred_element_type=jnp.float32) + b_ref[...]


def _mid(partial1, r1, wcat, bcat):
    m = wcat.shape[0]
    return pl.pallas_call(
        _mid_body,
        grid=(N // _BLK,),
        in_specs=[
            pl.BlockSpec((2, _BLK, H), lambda i: (0, i, 0)),
            pl.BlockSpec((_BLK, H), lambda i: (i, 0)),
            pl.BlockSpec((m, H), lambda i: (0, 0)),
            pl.BlockSpec((1, m), lambda i: (0, 0)),
        ],
        out_specs=pl.BlockSpec((_BLK, m), lambda i: (i, 0)),
        out_shape=jax.ShapeDtypeStruct((N, m), jnp.float32),
    )(partial1, r1, wcat, bcat)


def _final_body(pa_ref, r_ref, o_ref):
    o = pa_ref[0] + pa_ref[1] + r_ref[...]
    mask = lax.broadcasted_iota(jnp.int32, o.shape, 1) < C
    neg = jnp.where(mask, o, -jnp.inf)
    m = jnp.max(neg, axis=1, keepdims=True)
    e = jnp.where(mask, jnp.exp(o - m), 0.0)
    ssum = jnp.sum(e, axis=1, keepdims=True)
    o_ref[...] = o - m - jnp.log(ssum)


def _final(partial2, r2):
    return pl.pallas_call(
        _final_body,
        grid=(N // _BLK,),
        in_specs=[
            pl.BlockSpec((2, _BLK, H), lambda i: (0, i, 0)),
            pl.BlockSpec((_BLK, H), lambda i: (i, 0)),
        ],
        out_specs=pl.BlockSpec((_BLK, H), lambda i: (i, 0)),
        out_shape=jax.ShapeDtypeStruct((N, H), jnp.float32),
    )(partial2, r2)


def kernel(x, edge_index, W1_rel, b1, W1_root, W2_rel, b2, W2_root):
    src = edge_index[0]
    dst = edge_index[1]
    pad = _EPW_PAD - _EPW
    srcw = jnp.pad(src.reshape(_NW, _EPW), ((0, 0), (0, pad)))
    dstw = jnp.pad(dst.reshape(_NW, _EPW), ((0, 0), (0, pad)),
                   constant_values=N)
    srcw = srcw.reshape(_NW, _K, _B)
    dstw = dstw.reshape(_NW, _K, _B)
    zrows = jnp.zeros((_ZROWS, H), jnp.float32)

    wcat1 = jnp.concatenate([W1_rel, W1_root], axis=0)
    bcat1 = jnp.concatenate([jnp.zeros_like(b1), b1]).reshape(1, 2 * H)
    out1 = _proj(x, wcat1, bcat1)
    p1 = out1[:, :H]
    r1 = out1[:, H:]
    partial1 = _seg_partial(p1, srcw, dstw, zrows)[:, :N]

    w2rel = jnp.pad(W2_rel, ((0, H - C), (0, 0)))
    w2root = jnp.pad(W2_root, ((0, H - C), (0, 0)))
    wcat2 = jnp.concatenate([w2rel, w2root], axis=0)
    b2p = jnp.pad(b2, (0, H - C))
    bcat2 = jnp.concatenate([jnp.zeros_like(b2p), b2p]).reshape(1, 2 * H)
    out2 = _mid(partial1, r1, wcat2, bcat2)
    p2 = out2[:, :H]
    r2 = out2[:, H:]
    partial2 = _seg_partial(p2, srcw, dstw, zrows)[:, :N]

    out16 = _final(partial2, r2)
    return out16[:, :C]


# speedup vs baseline: 1.0926x; 1.0926x over previous
import functools

import jax
import jax.numpy as jnp
from jax import lax
from jax.experimental import pallas as pl
from jax.experimental.pallas import tpu as pltpu
from jax.experimental.pallas import tpu_sc as plsc

N = 10000
E = 320000
D = 128
H = 16
C = 10

_NW = 32
_EPW = E // _NW
_B = 256
_NBUF = 1
_K = -(-_EPW // _B)
_EPW_PAD = _K * _B
_ACC_ROWS = 10112
_ZROWS = _ACC_ROWS // 16
_OROWS = _ACC_ROWS // 16


def _seg_body(table_hbm, src_hbm, dst_hbm, zeros_hbm, out_hbm,
              src_v, dst_v, rows_v, acc_sh, *sems):
    gsems = sems[:_NBUF]
    ssems = sems[_NBUF:]
    c = lax.axis_index("c")
    s = lax.axis_index("s")
    wid = c * 16 + s
    pltpu.sync_copy(src_hbm.at[wid], src_v)
    pltpu.sync_copy(dst_hbm.at[wid], dst_v)
    pltpu.sync_copy(zeros_hbm, acc_sh.at[pl.ds(s * _ZROWS, _ZROWS)])
    plsc.subcore_barrier()

    def chunk(j, carry):
        pltpu.async_copy(table_hbm.at[src_v.at[j]], rows_v, gsems[0]).wait()
        pltpu.sync_copy(rows_v, acc_sh.at[dst_v.at[j]], add=True)
        return carry

    lax.fori_loop(0, _K, chunk, 0)
    plsc.subcore_barrier()
    pltpu.sync_copy(acc_sh.at[pl.ds(s * _OROWS, _OROWS)],
                    out_hbm.at[c, pl.ds(s * _OROWS, _OROWS)])


_seg_partial = functools.partial(
    pl.kernel,
    out_type=jax.ShapeDtypeStruct((2, _ACC_ROWS, H), jnp.float32),
    mesh=plsc.VectorSubcoreMesh(core_axis_name="c", subcore_axis_name="s"),
    compiler_params=pltpu.CompilerParams(use_tc_tiling_on_sc=False),
    scratch_types=[
        pltpu.VMEM((_K, _B), jnp.int32),
        pltpu.VMEM((_K, _B), jnp.int32),
        pltpu.VMEM((_B, H), jnp.float32),
        pltpu.VMEM_SHARED((_ACC_ROWS, H), jnp.float32),
    ] + [pltpu.SemaphoreType.DMA] * 2,
)(_seg_body)


_BLK = 2000


def _proj_body(x_ref, w_ref, b_ref, o_ref):
    o_ref[...] = lax.dot_general(
        x_ref[...], w_ref[...], (((1,), (1,)), ((), ())),
        preferred_element_type=jnp.float32) + b_ref[...]


def _proj(xx, wcat, bcat):
    n, d = xx.shape
    m = wcat.shape[0]
    return pl.pallas_call(
        _proj_body,
        grid=(n // _BLK,),
        in_specs=[
            pl.BlockSpec((_BLK, d), lambda i: (i, 0)),
            pl.BlockSpec((m, d), lambda i: (0, 0)),
            pl.BlockSpec((1, m), lambda i: (0, 0)),
        ],
        out_specs=pl.BlockSpec((_BLK, m), lambda i: (i, 0)),
        out_shape=jax.ShapeDtypeStruct((n, m), jnp.float32),
    )(xx, wcat, bcat)


def _mid_body(pa_ref, r_ref, w_ref, b_ref, o_ref):
    agg = pa_ref[0] + pa_ref[1]
    h = jnp.maximum(agg + r_ref[...], 0.0)
    o_ref[...] = lax.dot_general(
        h, w_ref[...], (((1,), (1,)), ((), ())),
        preferred_element_type=jnp.float32) + b_ref[...]


def _mid(partial1, r1, wcat, bcat):
    m = wcat.shape[0]
    return pl.pallas_call(
        _mid_body,
        grid=(N // _BLK,),
        in_specs=[
            pl.BlockSpec((2, _BLK, H), lambda i: (0, i, 0)),
            pl.BlockSpec((_BLK, H), lambda i: (i, 0)),
            pl.BlockSpec((m, H), lambda i: (0, 0)),
            pl.BlockSpec((1, m), lambda i: (0, 0)),
        ],
        out_specs=pl.BlockSpec((_BLK, m), lambda i: (i, 0)),
        out_shape=jax.ShapeDtypeStruct((N, m), jnp.float32),
    )(partial1, r1, wcat, bcat)


def _final_body(pa_ref, r_ref, o_ref):
    o = pa_ref[0] + pa_ref[1] + r_ref[...]
    mask = lax.broadcasted_iota(jnp.int32, o.shape, 1) < C
    neg = jnp.where(mask, o, -jnp.inf)
    m = jnp.max(neg, axis=1, keepdims=True)
    e = jnp.where(mask, jnp.exp(o - m), 0.0)
    ssum = jnp.sum(e, axis=1, keepdims=True)
    o_ref[...] = o - m - jnp.log(ssum)


def _final(partial2, r2):
    return pl.pallas_call(
        _final_body,
        grid=(N // _BLK,),
        in_specs=[
            pl.BlockSpec((2, _BLK, H), lambda i: (0, i, 0)),
            pl.BlockSpec((_BLK, H), lambda i: (i, 0)),
        ],
        out_specs=pl.BlockSpec((_BLK, H), lambda i: (i, 0)),
        out_shape=jax.ShapeDtypeStruct((N, H), jnp.float32),
    )(partial2, r2)


def kernel(x, edge_index, W1_rel, b1, W1_root, W2_rel, b2, W2_root):
    src = edge_index[0]
    dst = edge_index[1]
    pad = _EPW_PAD - _EPW
    srcw = jnp.pad(src.reshape(_NW, _EPW), ((0, 0), (0, pad)))
    dstw = jnp.pad(dst.reshape(_NW, _EPW), ((0, 0), (0, pad)),
                   constant_values=N)
    srcw = srcw.reshape(_NW, _K, _B)
    dstw = dstw.reshape(_NW, _K, _B)
    zrows = jnp.zeros((_ZROWS, H), jnp.float32)

    wcat1 = jnp.concatenate([W1_rel, W1_root], axis=0)
    bcat1 = jnp.concatenate([jnp.zeros_like(b1), b1]).reshape(1, 2 * H)
    out1 = _proj(x, wcat1, bcat1)
    p1 = out1[:, :H]
    r1 = out1[:, H:]
    partial1 = _seg_partial(p1, srcw, dstw, zrows)[:, :N]

    w2rel = jnp.pad(W2_rel, ((0, H - C), (0, 0)))
    w2root = jnp.pad(W2_root, ((0, H - C), (0, 0)))
    wcat2 = jnp.concatenate([w2rel, w2root], axis=0)
    b2p = jnp.pad(b2, (0, H - C))
    bcat2 = jnp.concatenate([jnp.zeros_like(b2p), b2p]).reshape(1, 2 * H)
    out2 = _mid(partial1, r1, wcat2, bcat2)
    p2 = out2[:, :H]
    r2 = out2[:, H:]
    partial2 = _seg_partial(p2, srcw, dstw, zrows)[:, :N]

    out16 = _final(partial2, r2)
    return out16[:, :C]


# speedup vs baseline: 1.2575x; 1.1509x over previous
import functools

import jax
import jax.numpy as jnp
from jax import lax
from jax.experimental import pallas as pl
from jax.experimental.pallas import tpu as pltpu
from jax.experimental.pallas import tpu_sc as plsc

N = 10000
E = 320000
D = 128
H = 16
C = 10

_NW = 32
_EPW = E // _NW
_B = 512
_K = -(-_EPW // _B)
_EPW_PAD = _K * _B
_ACC_ROWS = 10112
_ZROWS = _ACC_ROWS // 16
_OROWS = 632
_OLAST = N - 15 * _OROWS


def _seg_body(table_hbm, src_hbm, dst_hbm, zeros_hbm, out_hbm,
              src_v, dst_v, rows_v, acc_sh, gsem):
    c = lax.axis_index("c")
    s = lax.axis_index("s")
    wid = c * 16 + s
    pltpu.sync_copy(src_hbm.at[wid], src_v)
    pltpu.sync_copy(dst_hbm.at[wid], dst_v)
    pltpu.sync_copy(zeros_hbm, acc_sh.at[pl.ds(s * _ZROWS, _ZROWS)])
    plsc.subcore_barrier()

    def chunk(j, carry):
        pltpu.async_copy(table_hbm.at[src_v.at[j]], rows_v, gsem).wait()
        pltpu.sync_copy(rows_v, acc_sh.at[dst_v.at[j]], add=True)
        return carry

    lax.fori_loop(0, _K, chunk, 0)
    plsc.subcore_barrier()
    @pl.when(s < 15)
    def _():
        pltpu.sync_copy(acc_sh.at[pl.ds(s * _OROWS, _OROWS)],
                        out_hbm.at[c, pl.ds(s * _OROWS, _OROWS)])

    @pl.when(s == 15)
    def _():
        pltpu.sync_copy(acc_sh.at[pl.ds(15 * _OROWS, _OLAST)],
                        out_hbm.at[c, pl.ds(15 * _OROWS, _OLAST)])


_seg_partial = functools.partial(
    pl.kernel,
    out_type=jax.ShapeDtypeStruct((2, N, H), jnp.float32),
    mesh=plsc.VectorSubcoreMesh(core_axis_name="c", subcore_axis_name="s"),
    compiler_params=pltpu.CompilerParams(use_tc_tiling_on_sc=False),
    scratch_types=[
        pltpu.VMEM((_K, _B), jnp.int32),
        pltpu.VMEM((_K, _B), jnp.int32),
        pltpu.VMEM((_B, H), jnp.float32),
        pltpu.VMEM_SHARED((_ACC_ROWS, H), jnp.float32),
        pltpu.SemaphoreType.DMA,
    ],
)(_seg_body)


_BLK = 2000


def _proj_body(x_ref, w_ref, b_ref, o_ref):
    o_ref[...] = lax.dot_general(
        x_ref[...], w_ref[...], (((1,), (1,)), ((), ())),
        preferred_element_type=jnp.float32) + b_ref[...]


def _proj(xx, wcat, bcat):
    n, d = xx.shape
    m = wcat.shape[0]
    return pl.pallas_call(
        _proj_body,
        grid=(n // _BLK,),
        in_specs=[
            pl.BlockSpec((_BLK, d), lambda i: (i, 0)),
            pl.BlockSpec((m, d), lambda i: (0, 0)),
            pl.BlockSpec((1, m), lambda i: (0, 0)),
        ],
        out_specs=pl.BlockSpec((_BLK, m), lambda i: (i, 0)),
        out_shape=jax.ShapeDtypeStruct((n, m), jnp.float32),
    )(xx, wcat, bcat)


def _mid_body(pa_ref, r_ref, w_ref, b_ref, o_ref):
    agg = pa_ref[0] + pa_ref[1]
    h = jnp.maximum(agg + r_ref[...], 0.0)
    o_ref[...] = lax.dot_general(
        h, w_ref[...], (((1,), (1,)), ((), ())),
        preferred_element_type=jnp.float32) + b_ref[...]


def _mid(partial1, r1, wcat, bcat):
    m = wcat.shape[0]
    return pl.pallas_call(
        _mid_body,
        grid=(N // _BLK,),
        in_specs=[
            pl.BlockSpec((2, _BLK, H), lambda i: (0, i, 0)),
            pl.BlockSpec((_BLK, H), lambda i: (i, 0)),
            pl.BlockSpec((m, H), lambda i: (0, 0)),
            pl.BlockSpec((1, m), lambda i: (0, 0)),
        ],
        out_specs=pl.BlockSpec((_BLK, m), lambda i: (i, 0)),
        out_shape=jax.ShapeDtypeStruct((N, m), jnp.float32),
    )(partial1, r1, wcat, bcat)


def _final_body(pa_ref, r_ref, o_ref):
    o = pa_ref[0] + pa_ref[1] + r_ref[...]
    mask = lax.broadcasted_iota(jnp.int32, o.shape, 1) < C
    neg = jnp.where(mask, o, -jnp.inf)
    m = jnp.max(neg, axis=1, keepdims=True)
    e = jnp.where(mask, jnp.exp(o - m), 0.0)
    ssum = jnp.sum(e, axis=1, keepdims=True)
    o_ref[...] = o - m - jnp.log(ssum)


def _final(partial2, r2):
    return pl.pallas_call(
        _final_body,
        grid=(N // _BLK,),
        in_specs=[
            pl.BlockSpec((2, _BLK, H), lambda i: (0, i, 0)),
            pl.BlockSpec((_BLK, H), lambda i: (i, 0)),
        ],
        out_specs=pl.BlockSpec((_BLK, H), lambda i: (i, 0)),
        out_shape=jax.ShapeDtypeStruct((N, H), jnp.float32),
    )(partial2, r2)


def kernel(x, edge_index, W1_rel, b1, W1_root, W2_rel, b2, W2_root):
    src = edge_index[0]
    dst = edge_index[1]
    pad = _EPW_PAD - _EPW
    srcw = jnp.pad(src.reshape(_NW, _EPW), ((0, 0), (0, pad)))
    dstw = jnp.pad(dst.reshape(_NW, _EPW), ((0, 0), (0, pad)),
                   constant_values=N)
    srcw = srcw.reshape(_NW, _K, _B)
    dstw = dstw.reshape(_NW, _K, _B)
    zrows = jnp.zeros((_ZROWS, H), jnp.float32)

    wcat1 = jnp.concatenate([W1_rel, W1_root], axis=0)
    bcat1 = jnp.concatenate([jnp.zeros_like(b1), b1]).reshape(1, 2 * H)
    out1 = _proj(x, wcat1, bcat1)
    p1 = out1[:, :H]
    r1 = out1[:, H:]
    partial1 = _seg_partial(p1, srcw, dstw, zrows)

    w2rel = jnp.pad(W2_rel, ((0, H - C), (0, 0)))
    w2root = jnp.pad(W2_root, ((0, H - C), (0, 0)))
    wcat2 = jnp.concatenate([w2rel, w2root], axis=0)
    b2p = jnp.pad(b2, (0, H - C))
    bcat2 = jnp.concatenate([jnp.zeros_like(b2p), b2p]).reshape(1, 2 * H)
    out2 = _mid(partial1, r1, wcat2, bcat2)
    p2 = out2[:, :H]
    r2 = out2[:, H:]
    partial2 = _seg_partial(p2, srcw, dstw, zrows)

    out16 = _final(partial2, r2)
    return out16[:, :C]


# speedup vs baseline: 1.9526x; 1.5528x over previous
import functools

import jax
import jax.numpy as jnp
from jax import lax
from jax.experimental import pallas as pl
from jax.experimental.pallas import tpu as pltpu
from jax.experimental.pallas import tpu_sc as plsc

N = 10000
E = 320000
D = 128
H = 16
C = 10

_NW = 32
_EPW = E // _NW
_B = 1000
_K = _EPW // _B
_ACC_ROWS = 10112
_ZROWS = _ACC_ROWS // 16
_OROWS = 632
_OLAST = N - 15 * _OROWS


def _seg_body(table_hbm, src_hbm, dst_hbm, zeros_hbm, out_hbm,
              src_v, dst_v, rows_v, acc_sh, gsem):
    c = lax.axis_index("c")
    s = lax.axis_index("s")
    wid = c * 16 + s
    base = wid * _EPW
    pltpu.sync_copy(src_hbm.at[pl.ds(base, _EPW)], src_v)
    pltpu.sync_copy(dst_hbm.at[pl.ds(base, _EPW)], dst_v)
    pltpu.sync_copy(zeros_hbm, acc_sh.at[pl.ds(s * _ZROWS, _ZROWS)])
    plsc.subcore_barrier()

    def chunk(j, carry):
        o = pl.multiple_of(j * _B, _B)
        pltpu.async_copy(table_hbm.at[src_v.at[pl.ds(o, _B)]], rows_v,
                         gsem).wait()
        pltpu.sync_copy(rows_v, acc_sh.at[dst_v.at[pl.ds(o, _B)]], add=True)
        return carry

    lax.fori_loop(0, _K, chunk, 0)
    plsc.subcore_barrier()
    @pl.when(s < 15)
    def _():
        pltpu.sync_copy(acc_sh.at[pl.ds(s * _OROWS, _OROWS)],
                        out_hbm.at[c, pl.ds(s * _OROWS, _OROWS)])

    @pl.when(s == 15)
    def _():
        pltpu.sync_copy(acc_sh.at[pl.ds(15 * _OROWS, _OLAST)],
                        out_hbm.at[c, pl.ds(15 * _OROWS, _OLAST)])


_seg_partial = functools.partial(
    pl.kernel,
    out_type=jax.ShapeDtypeStruct((2, N, H), jnp.float32),
    mesh=plsc.VectorSubcoreMesh(core_axis_name="c", subcore_axis_name="s"),
    compiler_params=pltpu.CompilerParams(use_tc_tiling_on_sc=False),
    scratch_types=[
        pltpu.VMEM((_EPW,), jnp.int32),
        pltpu.VMEM((_EPW,), jnp.int32),
        pltpu.VMEM((_B, H), jnp.float32),
        pltpu.VMEM_SHARED((_ACC_ROWS, H), jnp.float32),
        pltpu.SemaphoreType.DMA,
    ],
)(_seg_body)


_BLK = 2000


def _proj_body(x_ref, w_ref, b_ref, o_ref):
    o_ref[...] = lax.dot_general(
        x_ref[...], w_ref[...], (((1,), (1,)), ((), ())),
        preferred_element_type=jnp.float32) + b_ref[...]


def _proj(xx, wcat, bcat):
    n, d = xx.shape
    m = wcat.shape[0]
    return pl.pallas_call(
        _proj_body,
        grid=(n // _BLK,),
        in_specs=[
            pl.BlockSpec((_BLK, d), lambda i: (i, 0)),
            pl.BlockSpec((m, d), lambda i: (0, 0)),
            pl.BlockSpec((1, m), lambda i: (0, 0)),
        ],
        out_specs=pl.BlockSpec((_BLK, m), lambda i: (i, 0)),
        out_shape=jax.ShapeDtypeStruct((n, m), jnp.float32),
    )(xx, wcat, bcat)


def _mid_body(pa_ref, r_ref, w_ref, b_ref, o_ref):
    agg = pa_ref[0] + pa_ref[1]
    h = jnp.maximum(agg + r_ref[...], 0.0)
    o_ref[...] = lax.dot_general(
        h, w_ref[...], (((1,), (1,)), ((), ())),
        preferred_element_type=jnp.float32) + b_ref[...]


def _mid(partial1, r1, wcat, bcat):
    m = wcat.shape[0]
    return pl.pallas_call(
        _mid_body,
        grid=(N // _BLK,),
        in_specs=[
            pl.BlockSpec((2, _BLK, H), lambda i: (0, i, 0)),
            pl.BlockSpec((_BLK, H), lambda i: (i, 0)),
            pl.BlockSpec((m, H), lambda i: (0, 0)),
            pl.BlockSpec((1, m), lambda i: (0, 0)),
        ],
        out_specs=pl.BlockSpec((_BLK, m), lambda i: (i, 0)),
        out_shape=jax.ShapeDtypeStruct((N, m), jnp.float32),
    )(partial1, r1, wcat, bcat)


def _final_body(pa_ref, r_ref, o_ref):
    o = pa_ref[0] + pa_ref[1] + r_ref[...]
    mask = lax.broadcasted_iota(jnp.int32, o.shape, 1) < C
    neg = jnp.where(mask, o, -jnp.inf)
    m = jnp.max(neg, axis=1, keepdims=True)
    e = jnp.where(mask, jnp.exp(o - m), 0.0)
    ssum = jnp.sum(e, axis=1, keepdims=True)
    o_ref[...] = o - m - jnp.log(ssum)


def _final(partial2, r2):
    return pl.pallas_call(
        _final_body,
        grid=(N // _BLK,),
        in_specs=[
            pl.BlockSpec((2, _BLK, H), lambda i: (0, i, 0)),
            pl.BlockSpec((_BLK, H), lambda i: (i, 0)),
        ],
        out_specs=pl.BlockSpec((_BLK, H), lambda i: (i, 0)),
        out_shape=jax.ShapeDtypeStruct((N, H), jnp.float32),
    )(partial2, r2)


def kernel(x, edge_index, W1_rel, b1, W1_root, W2_rel, b2, W2_root):
    src = edge_index[0]
    dst = edge_index[1]
    zrows = jnp.zeros((_ZROWS, H), jnp.float32)

    wcat1 = jnp.concatenate([W1_rel, W1_root], axis=0)
    bcat1 = jnp.concatenate([jnp.zeros_like(b1), b1]).reshape(1, 2 * H)
    out1 = _proj(x, wcat1, bcat1)
    p1 = out1[:, :H]
    r1 = out1[:, H:]
    partial1 = _seg_partial(p1, src, dst, zrows)

    w2rel = jnp.pad(W2_rel, ((0, H - C), (0, 0)))
    w2root = jnp.pad(W2_root, ((0, H - C), (0, 0)))
    wcat2 = jnp.concatenate([w2rel, w2root], axis=0)
    b2p = jnp.pad(b2, (0, H - C))
    bcat2 = jnp.concatenate([jnp.zeros_like(b2p), b2p]).reshape(1, 2 * H)
    out2 = _mid(partial1, r1, wcat2, bcat2)
    p2 = out2[:, :H]
    r2 = out2[:, H:]
    partial2 = _seg_partial(p2, src, dst, zrows)

    out16 = _final(partial2, r2)
    return out16[:, :C]


# speedup vs baseline: 2.1561x; 1.1042x over previous
import functools

import jax
import jax.numpy as jnp
from jax import lax
from jax.experimental import pallas as pl
from jax.experimental.pallas import tpu as pltpu
from jax.experimental.pallas import tpu_sc as plsc

N = 10000
E = 320000
D = 128
H = 16
C = 10

_NW = 32
_EPW = E // _NW
_B = 1000
_K = _EPW // _B
_ACC_ROWS = 10112
_ZROWS = _ACC_ROWS // 16
_OROWS = 632
_OLAST = N - 15 * _OROWS


def _seg_body(table_hbm, src_hbm, dst_hbm, zeros_hbm, out_hbm,
              src_v, dst_v, rows0_v, rows1_v, acc_sh, sem0, sem1, ssem):
    c = lax.axis_index("c")
    s = lax.axis_index("s")
    wid = c * 16 + s
    base = wid * _EPW
    cp_src = pltpu.async_copy(src_hbm.at[pl.ds(base, _EPW)], src_v, ssem)
    cp_dst = pltpu.async_copy(dst_hbm.at[pl.ds(base, _EPW)], dst_v, sem1)
    cp_z = pltpu.async_copy(zeros_hbm, acc_sh.at[pl.ds(s * _ZROWS, _ZROWS)],
                            sem0)
    cp_src.wait()
    bufs = (rows0_v, rows1_v)
    sems = (sem0, sem1)
    cp_z.wait()
    cp_dst.wait()
    g = pltpu.async_copy(table_hbm.at[src_v.at[pl.ds(0, _B)]], bufs[0],
                         sems[0])
    plsc.subcore_barrier()

    for j in range(_K):
        g.wait()
        if j + 1 < _K:
            nxt = (j + 1) % 2
            g = pltpu.async_copy(
                table_hbm.at[src_v.at[pl.ds((j + 1) * _B, _B)]],
                bufs[nxt], sems[nxt])
        pltpu.sync_copy(bufs[j % 2],
                        acc_sh.at[dst_v.at[pl.ds(j * _B, _B)]], add=True)
    plsc.subcore_barrier()
    @pl.when(s < 15)
    def _():
        pltpu.sync_copy(acc_sh.at[pl.ds(s * _OROWS, _OROWS)],
                        out_hbm.at[c, pl.ds(s * _OROWS, _OROWS)])

    @pl.when(s == 15)
    def _():
        pltpu.sync_copy(acc_sh.at[pl.ds(15 * _OROWS, _OLAST)],
                        out_hbm.at[c, pl.ds(15 * _OROWS, _OLAST)])


_seg_partial = functools.partial(
    pl.kernel,
    out_type=jax.ShapeDtypeStruct((2, N, H), jnp.float32),
    mesh=plsc.VectorSubcoreMesh(core_axis_name="c", subcore_axis_name="s"),
    compiler_params=pltpu.CompilerParams(use_tc_tiling_on_sc=False),
    scratch_types=[
        pltpu.VMEM((_EPW,), jnp.int32),
        pltpu.VMEM((_EPW,), jnp.int32),
        pltpu.VMEM((_B, H), jnp.float32),
        pltpu.VMEM((_B, H), jnp.float32),
        pltpu.VMEM_SHARED((_ACC_ROWS, H), jnp.float32),
        pltpu.SemaphoreType.DMA,
        pltpu.SemaphoreType.DMA,
        pltpu.SemaphoreType.DMA,
    ],
)(_seg_body)


_BLK = 2000


def _proj_body(x_ref, w_ref, b_ref, o_ref):
    o_ref[...] = lax.dot_general(
        x_ref[...], w_ref[...], (((1,), (1,)), ((), ())),
        preferred_element_type=jnp.float32) + b_ref[...]


def _proj(xx, wcat, bcat):
    n, d = xx.shape
    m = wcat.shape[0]
    return pl.pallas_call(
        _proj_body,
        grid=(n // _BLK,),
        in_specs=[
            pl.BlockSpec((_BLK, d), lambda i: (i, 0)),
            pl.BlockSpec((m, d), lambda i: (0, 0)),
            pl.BlockSpec((1, m), lambda i: (0, 0)),
        ],
        out_specs=pl.BlockSpec((_BLK, m), lambda i: (i, 0)),
        out_shape=jax.ShapeDtypeStruct((n, m), jnp.float32),
    )(xx, wcat, bcat)


def _mid_body(pa_ref, r_ref, w_ref, b_ref, o_ref):
    agg = pa_ref[0] + pa_ref[1]
    h = jnp.maximum(agg + r_ref[...], 0.0)
    o_ref[...] = lax.dot_general(
        h, w_ref[...], (((1,), (1,)), ((), ())),
        preferred_element_type=jnp.float32) + b_ref[...]


def _mid(partial1, r1, wcat, bcat):
    m = wcat.shape[0]
    return pl.pallas_call(
        _mid_body,
        grid=(N // _BLK,),
        in_specs=[
            pl.BlockSpec((2, _BLK, H), lambda i: (0, i, 0)),
            pl.BlockSpec((_BLK, H), lambda i: (i, 0)),
            pl.BlockSpec((m, H), lambda i: (0, 0)),
            pl.BlockSpec((1, m), lambda i: (0, 0)),
        ],
        out_specs=pl.BlockSpec((_BLK, m), lambda i: (i, 0)),
        out_shape=jax.ShapeDtypeStruct((N, m), jnp.float32),
    )(partial1, r1, wcat, bcat)


def _final_body(pa_ref, r_ref, o_ref):
    o = pa_ref[0] + pa_ref[1] + r_ref[...]
    mask = lax.broadcasted_iota(jnp.int32, o.shape, 1) < C
    neg = jnp.where(mask, o, -jnp.inf)
    m = jnp.max(neg, axis=1, keepdims=True)
    e = jnp.where(mask, jnp.exp(o - m), 0.0)
    ssum = jnp.sum(e, axis=1, keepdims=True)
    o_ref[...] = o - m - jnp.log(ssum)


def _final(partial2, r2):
    return pl.pallas_call(
        _final_body,
        grid=(N // _BLK,),
        in_specs=[
            pl.BlockSpec((2, _BLK, H), lambda i: (0, i, 0)),
            pl.BlockSpec((_BLK, H), lambda i: (i, 0)),
        ],
        out_specs=pl.BlockSpec((_BLK, H), lambda i: (i, 0)),
        out_shape=jax.ShapeDtypeStruct((N, H), jnp.float32),
    )(partial2, r2)


def kernel(x, edge_index, W1_rel, b1, W1_root, W2_rel, b2, W2_root):
    src = edge_index[0]
    dst = edge_index[1]
    zrows = jnp.zeros((_ZROWS, H), jnp.float32)

    wcat1 = jnp.concatenate([W1_rel, W1_root], axis=0)
    bcat1 = jnp.concatenate([jnp.zeros_like(b1), b1]).reshape(1, 2 * H)
    out1 = _proj(x, wcat1, bcat1)
    p1 = out1[:, :H]
    r1 = out1[:, H:]
    partial1 = _seg_partial(p1, src, dst, zrows)

    w2rel = jnp.pad(W2_rel, ((0, H - C), (0, 0)))
    w2root = jnp.pad(W2_root, ((0, H - C), (0, 0)))
    wcat2 = jnp.concatenate([w2rel, w2root], axis=0)
    b2p = jnp.pad(b2, (0, H - C))
    bcat2 = jnp.concatenate([jnp.zeros_like(b2p), b2p]).reshape(1, 2 * H)
    out2 = _mid(partial1, r1, wcat2, bcat2)
    p2 = out2[:, :H]
    r2 = out2[:, H:]
    partial2 = _seg_partial(p2, src, dst, zrows)

    out16 = _final(partial2, r2)
    return out16[:, :C]


# speedup vs baseline: 2.1745x; 1.0085x over previous
import functools

import jax
import jax.numpy as jnp
from jax import lax
from jax.experimental import pallas as pl
from jax.experimental.pallas import tpu as pltpu
from jax.experimental.pallas import tpu_sc as plsc

N = 10000
E = 320000
D = 128
H = 16
C = 10

_NW = 32
_EPW = E // _NW
_B = 1000
_K = _EPW // _B
_ACC_ROWS = 10112
_ZROWS = _ACC_ROWS // 16
_OROWS = 632
_OLAST = N - 15 * _OROWS


def _seg_body(table_hbm, src_hbm, dst_hbm, zeros_hbm, out_hbm,
              src_v, dst_v, rows0_v, rows1_v, acc_sh, sem0, sem1, ssem):
    c = lax.axis_index("c")
    s = lax.axis_index("s")
    wid = c * 16 + s
    base = wid * _EPW
    cp_src = pltpu.async_copy(src_hbm.at[pl.ds(base, _EPW)], src_v, ssem)
    cp_dst = pltpu.async_copy(dst_hbm.at[pl.ds(base, _EPW)], dst_v, sem1)
    cp_z = pltpu.async_copy(zeros_hbm, acc_sh.at[pl.ds(s * _ZROWS, _ZROWS)],
                            sem0)
    cp_src.wait()
    bufs = (rows0_v, rows1_v)
    sems = (sem0, sem1)
    cp_z.wait()
    cp_dst.wait()
    g = pltpu.async_copy(table_hbm.at[src_v.at[pl.ds(0, _B)]], bufs[0],
                         sems[0])
    plsc.subcore_barrier()

    for j in range(_K):
        g.wait()
        if j + 1 < _K:
            nxt = (j + 1) % 2
            g = pltpu.async_copy(
                table_hbm.at[src_v.at[pl.ds((j + 1) * _B, _B)]],
                bufs[nxt], sems[nxt])
        pltpu.sync_copy(bufs[j % 2],
                        acc_sh.at[dst_v.at[pl.ds(j * _B, _B)]], add=True)
    plsc.subcore_barrier()
    @pl.when(s < 15)
    def _():
        pltpu.sync_copy(acc_sh.at[pl.ds(s * _OROWS, _OROWS)],
                        out_hbm.at[c, pl.ds(s * _OROWS, _OROWS)])

    @pl.when(s == 15)
    def _():
        pltpu.sync_copy(acc_sh.at[pl.ds(15 * _OROWS, _OLAST)],
                        out_hbm.at[c, pl.ds(15 * _OROWS, _OLAST)])


_seg_partial = functools.partial(
    pl.kernel,
    out_type=jax.ShapeDtypeStruct((2, N, H), jnp.float32),
    mesh=plsc.VectorSubcoreMesh(core_axis_name="c", subcore_axis_name="s"),
    compiler_params=pltpu.CompilerParams(use_tc_tiling_on_sc=False),
    scratch_types=[
        pltpu.VMEM((_EPW,), jnp.int32),
        pltpu.VMEM((_EPW,), jnp.int32),
        pltpu.VMEM((_B, H), jnp.float32),
        pltpu.VMEM((_B, H), jnp.float32),
        pltpu.VMEM_SHARED((_ACC_ROWS, H), jnp.float32),
        pltpu.SemaphoreType.DMA,
        pltpu.SemaphoreType.DMA,
        pltpu.SemaphoreType.DMA,
    ],
)(_seg_body)


def _fused_body(p_hbm, r_hbm, src_hbm, dst_hbm, zeros_hbm,
                out_hbm, h_hbm,
                src_v, dst_v, rows0_v, rows1_v, rbuf_v, acc_sh,
                sem0, sem1, ssem):
    c = lax.axis_index("c")
    s = lax.axis_index("s")
    wid = c * 16 + s
    base = wid * _EPW
    cp_src = pltpu.async_copy(src_hbm.at[pl.ds(base, _EPW)], src_v, ssem)
    cp_dst = pltpu.async_copy(dst_hbm.at[pl.ds(base, _EPW)], dst_v, sem1)
    cp_z = pltpu.async_copy(zeros_hbm, acc_sh.at[pl.ds(s * _ZROWS, _ZROWS)],
                            sem0)

    def do_h(off, nrows):
        pltpu.sync_copy(p_hbm.at[0, pl.ds(off, nrows)],
                        rows0_v.at[pl.ds(0, nrows)])
        pltpu.sync_copy(p_hbm.at[1, pl.ds(off, nrows)],
                        rows1_v.at[pl.ds(0, nrows)])
        pltpu.sync_copy(r_hbm.at[pl.ds(off, nrows)],
                        rbuf_v.at[pl.ds(0, nrows)])

        def row(i, carry):
            rows0_v[i] = jnp.maximum(rows0_v[i] + rows1_v[i] + rbuf_v[i],
                                     0.0)
            return carry

        lax.fori_loop(0, nrows, row, 0)
        pltpu.sync_copy(rows0_v.at[pl.ds(0, nrows)],
                        h_hbm.at[c, pl.ds(off, nrows)])

    @pl.when(s < 15)
    def _():
        do_h(s * _OROWS, _OROWS)

    @pl.when(s == 15)
    def _():
        do_h(15 * _OROWS, _OLAST)

    cp_src.wait()
    cp_z.wait()
    cp_dst.wait()
    plsc.subcore_barrier()

    g = pltpu.async_copy(h_hbm.at[c].at[src_v.at[pl.ds(0, _B)]], rows0_v,
                         sem0)
    for j in range(_K):
        g.wait()
        if j + 1 < _K:
            nxt = (j + 1) % 2
            g = pltpu.async_copy(
                h_hbm.at[c].at[src_v.at[pl.ds((j + 1) * _B, _B)]],
                (rows0_v, rows1_v)[nxt], (sem0, sem1)[nxt])
        pltpu.sync_copy((rows0_v, rows1_v)[j % 2],
                        acc_sh.at[dst_v.at[pl.ds(j * _B, _B)]], add=True)
    plsc.subcore_barrier()

    @pl.when(s < 15)
    def _():
        pltpu.sync_copy(acc_sh.at[pl.ds(s * _OROWS, _OROWS)],
                        out_hbm.at[c, pl.ds(s * _OROWS, _OROWS)])

    @pl.when(s == 15)
    def _():
        pltpu.sync_copy(acc_sh.at[pl.ds(15 * _OROWS, _OLAST)],
                        out_hbm.at[c, pl.ds(15 * _OROWS, _OLAST)])


_seg_fused = functools.partial(
    pl.kernel,
    out_type=(jax.ShapeDtypeStruct((2, N, H), jnp.float32),
              jax.ShapeDtypeStruct((2, N, H), jnp.float32)),
    mesh=plsc.VectorSubcoreMesh(core_axis_name="c", subcore_axis_name="s"),
    compiler_params=pltpu.CompilerParams(use_tc_tiling_on_sc=False),
    scratch_types=[
        pltpu.VMEM((_EPW,), jnp.int32),
        pltpu.VMEM((_EPW,), jnp.int32),
        pltpu.VMEM((_B, H), jnp.float32),
        pltpu.VMEM((_B, H), jnp.float32),
        pltpu.VMEM((_OROWS, H), jnp.float32),
        pltpu.VMEM_SHARED((_ACC_ROWS, H), jnp.float32),
        pltpu.SemaphoreType.DMA,
        pltpu.SemaphoreType.DMA,
        pltpu.SemaphoreType.DMA,
    ],
)(_fused_body)


_BLK = 2000


def _proj_body(x_ref, w_ref, b_ref, o_ref):
    o_ref[...] = lax.dot_general(
        x_ref[...], w_ref[...], (((1,), (1,)), ((), ())),
        preferred_element_type=jnp.float32) + b_ref[...]


def _proj(xx, wcat, bcat):
    n, d = xx.shape
    m = wcat.shape[0]
    return pl.pallas_call(
        _proj_body,
        grid=(n // _BLK,),
        in_specs=[
            pl.BlockSpec((_BLK, d), lambda i: (i, 0)),
            pl.BlockSpec((m, d), lambda i: (0, 0)),
            pl.BlockSpec((1, m), lambda i: (0, 0)),
        ],
        out_specs=pl.BlockSpec((_BLK, m), lambda i: (i, 0)),
        out_shape=jax.ShapeDtypeStruct((n, m), jnp.float32),
    )(xx, wcat, bcat)


def _final_body(pa_ref, h_ref, wrel_ref, wroot_ref, b_ref, o_ref):
    agg = pa_ref[0] + pa_ref[1]
    o = lax.dot_general(
        agg, wrel_ref[...], (((1,), (1,)), ((), ())),
        preferred_element_type=jnp.float32)
    o = o + lax.dot_general(
        h_ref[...], wroot_ref[...], (((1,), (1,)), ((), ())),
        preferred_element_type=jnp.float32)
    o = o + b_ref[...]
    mask = lax.broadcasted_iota(jnp.int32, o.shape, 1) < C
    neg = jnp.where(mask, o, -jnp.inf)
    m = jnp.max(neg, axis=1, keepdims=True)
    e = jnp.where(mask, jnp.exp(o - m), 0.0)
    ssum = jnp.sum(e, axis=1, keepdims=True)
    o_ref[...] = o - m - jnp.log(ssum)


def _final(partial2, h, wrel, wroot, b):
    return pl.pallas_call(
        _final_body,
        grid=(N // _BLK,),
        in_specs=[
            pl.BlockSpec((2, _BLK, H), lambda i: (0, i, 0)),
            pl.BlockSpec((_BLK, H), lambda i: (i, 0)),
            pl.BlockSpec((H, H), lambda i: (0, 0)),
            pl.BlockSpec((H, H), lambda i: (0, 0)),
            pl.BlockSpec((1, H), lambda i: (0, 0)),
        ],
        out_specs=pl.BlockSpec((_BLK, H), lambda i: (i, 0)),
        out_shape=jax.ShapeDtypeStruct((N, H), jnp.float32),
    )(partial2, h, wrel, wroot, b)


def kernel(x, edge_index, W1_rel, b1, W1_root, W2_rel, b2, W2_root):
    src = edge_index[0]
    dst = edge_index[1]
    zrows = jnp.zeros((_ZROWS, H), jnp.float32)

    wcat1 = jnp.concatenate([W1_rel, W1_root], axis=0)
    bcat1 = jnp.concatenate([jnp.zeros_like(b1), b1]).reshape(1, 2 * H)
    out1 = _proj(x, wcat1, bcat1)
    p1 = out1[:, :H]
    r1 = out1[:, H:]
    partial1 = _seg_partial(p1, src, dst, zrows)

    partial2, h2 = _seg_fused(partial1, r1, src, dst, zrows)

    w2rel = jnp.pad(W2_rel, ((0, H - C), (0, 0)))
    w2root = jnp.pad(W2_root, ((0, H - C), (0, 0)))
    b2p = jnp.pad(b2, (0, H - C)).reshape(1, H)
    out16 = _final(partial2, h2[0], w2rel, w2root, b2p)
    return out16[:, :C]


# speedup vs baseline: 2.2927x; 1.0544x over previous
import functools

import jax
import jax.numpy as jnp
from jax import lax
from jax.experimental import pallas as pl
from jax.experimental.pallas import tpu as pltpu
from jax.experimental.pallas import tpu_sc as plsc

N = 10000
E = 320000
D = 128
H = 16
C = 10

_NW = 32
_EPW = E // _NW
_B = 2000
_K = _EPW // _B
_ACC_ROWS = 10112
_ZROWS = _ACC_ROWS // 16
_OROWS = 632
_OLAST = N - 15 * _OROWS


def _seg_body(table_hbm, src_hbm, dst_hbm, zeros_hbm, out_hbm,
              src_v, dst_v, rows0_v, rows1_v, acc_sh, sem0, sem1, ssem):
    c = lax.axis_index("c")
    s = lax.axis_index("s")
    wid = c * 16 + s
    base = wid * _EPW
    cp_src = pltpu.async_copy(src_hbm.at[pl.ds(base, _EPW)], src_v, ssem)
    cp_dst = pltpu.async_copy(dst_hbm.at[pl.ds(base, _EPW)], dst_v, sem1)
    cp_z = pltpu.async_copy(zeros_hbm, acc_sh.at[pl.ds(s * _ZROWS, _ZROWS)],
                            sem0)
    cp_src.wait()
    bufs = (rows0_v, rows1_v)
    sems = (sem0, sem1)
    cp_z.wait()
    cp_dst.wait()
    g = pltpu.async_copy(table_hbm.at[src_v.at[pl.ds(0, _B)]], bufs[0],
                         sems[0])
    plsc.subcore_barrier()

    for j in range(_K):
        g.wait()
        if j + 1 < _K:
            nxt = (j + 1) % 2
            g = pltpu.async_copy(
                table_hbm.at[src_v.at[pl.ds((j + 1) * _B, _B)]],
                bufs[nxt], sems[nxt])
        pltpu.sync_copy(bufs[j % 2],
                        acc_sh.at[dst_v.at[pl.ds(j * _B, _B)]], add=True)
    plsc.subcore_barrier()
    @pl.when(s < 15)
    def _():
        pltpu.sync_copy(acc_sh.at[pl.ds(s * _OROWS, _OROWS)],
                        out_hbm.at[c, pl.ds(s * _OROWS, _OROWS)])

    @pl.when(s == 15)
    def _():
        pltpu.sync_copy(acc_sh.at[pl.ds(15 * _OROWS, _OLAST)],
                        out_hbm.at[c, pl.ds(15 * _OROWS, _OLAST)])


_seg_partial = functools.partial(
    pl.kernel,
    out_type=jax.ShapeDtypeStruct((2, N, H), jnp.float32),
    mesh=plsc.VectorSubcoreMesh(core_axis_name="c", subcore_axis_name="s"),
    compiler_params=pltpu.CompilerParams(use_tc_tiling_on_sc=False),
    scratch_types=[
        pltpu.VMEM((_EPW,), jnp.int32),
        pltpu.VMEM((_EPW,), jnp.int32),
        pltpu.VMEM((_B, H), jnp.float32),
        pltpu.VMEM((_B, H), jnp.float32),
        pltpu.VMEM_SHARED((_ACC_ROWS, H), jnp.float32),
        pltpu.SemaphoreType.DMA,
        pltpu.SemaphoreType.DMA,
        pltpu.SemaphoreType.DMA,
    ],
)(_seg_body)


def _fused_body(p_hbm, r_hbm, src_hbm, dst_hbm, zeros_hbm,
                out_hbm, h_hbm,
                src_v, dst_v, rows0_v, rows1_v, rbuf_v, acc_sh,
                sem0, sem1, ssem, hsem):
    c = lax.axis_index("c")
    s = lax.axis_index("s")
    wid = c * 16 + s
    base = wid * _EPW
    cp_src = pltpu.async_copy(src_hbm.at[pl.ds(base, _EPW)], src_v, ssem)
    cp_dst = pltpu.async_copy(dst_hbm.at[pl.ds(base, _EPW)], dst_v, sem1)
    cp_z = pltpu.async_copy(zeros_hbm, acc_sh.at[pl.ds(s * _ZROWS, _ZROWS)],
                            sem0)

    def do_h(off, nrows):
        c0 = pltpu.async_copy(p_hbm.at[0, pl.ds(off, nrows)],
                              rows0_v.at[pl.ds(0, nrows)], hsem)
        c1 = pltpu.async_copy(p_hbm.at[1, pl.ds(off, nrows)],
                              rows1_v.at[pl.ds(0, nrows)], hsem)
        c2 = pltpu.async_copy(r_hbm.at[pl.ds(off, nrows)],
                              rbuf_v.at[pl.ds(0, nrows)], hsem)
        c0.wait()
        c1.wait()
        c2.wait()

        def row(i, carry):
            rows0_v[i] = jnp.maximum(rows0_v[i] + rows1_v[i] + rbuf_v[i],
                                     0.0)
            return carry

        lax.fori_loop(0, nrows, row, 0)
        pltpu.sync_copy(rows0_v.at[pl.ds(0, nrows)],
                        h_hbm.at[c, pl.ds(off, nrows)])

    @pl.when(s < 15)
    def _():
        do_h(s * _OROWS, _OROWS)

    @pl.when(s == 15)
    def _():
        do_h(15 * _OROWS, _OLAST)

    cp_src.wait()
    cp_z.wait()
    cp_dst.wait()
    plsc.subcore_barrier()

    g = pltpu.async_copy(h_hbm.at[c].at[src_v.at[pl.ds(0, _B)]], rows0_v,
                         sem0)
    for j in range(_K):
        g.wait()
        if j + 1 < _K:
            nxt = (j + 1) % 2
            g = pltpu.async_copy(
                h_hbm.at[c].at[src_v.at[pl.ds((j + 1) * _B, _B)]],
                (rows0_v, rows1_v)[nxt], (sem0, sem1)[nxt])
        pltpu.sync_copy((rows0_v, rows1_v)[j % 2],
                        acc_sh.at[dst_v.at[pl.ds(j * _B, _B)]], add=True)
    plsc.subcore_barrier()

    @pl.when(s < 15)
    def _():
        pltpu.sync_copy(acc_sh.at[pl.ds(s * _OROWS, _OROWS)],
                        out_hbm.at[c, pl.ds(s * _OROWS, _OROWS)])

    @pl.when(s == 15)
    def _():
        pltpu.sync_copy(acc_sh.at[pl.ds(15 * _OROWS, _OLAST)],
                        out_hbm.at[c, pl.ds(15 * _OROWS, _OLAST)])


_seg_fused = functools.partial(
    pl.kernel,
    out_type=(jax.ShapeDtypeStruct((2, N, H), jnp.float32),
              jax.ShapeDtypeStruct((2, N, H), jnp.float32)),
    mesh=plsc.VectorSubcoreMesh(core_axis_name="c", subcore_axis_name="s"),
    compiler_params=pltpu.CompilerParams(use_tc_tiling_on_sc=False),
    scratch_types=[
        pltpu.VMEM((_EPW,), jnp.int32),
        pltpu.VMEM((_EPW,), jnp.int32),
        pltpu.VMEM((_B, H), jnp.float32),
        pltpu.VMEM((_B, H), jnp.float32),
        pltpu.VMEM((_OROWS, H), jnp.float32),
        pltpu.VMEM_SHARED((_ACC_ROWS, H), jnp.float32),
        pltpu.SemaphoreType.DMA,
        pltpu.SemaphoreType.DMA,
        pltpu.SemaphoreType.DMA,
        pltpu.SemaphoreType.DMA,
    ],
)(_fused_body)


_BLK = 2000


def _proj_body(x_ref, w_ref, b_ref, o_ref):
    o_ref[...] = lax.dot_general(
        x_ref[...], w_ref[...], (((1,), (1,)), ((), ())),
        preferred_element_type=jnp.float32) + b_ref[...]


def _proj(xx, wcat, bcat):
    n, d = xx.shape
    m = wcat.shape[0]
    return pl.pallas_call(
        _proj_body,
        grid=(n // _BLK,),
        in_specs=[
            pl.BlockSpec((_BLK, d), lambda i: (i, 0)),
            pl.BlockSpec((m, d), lambda i: (0, 0)),
            pl.BlockSpec((1, m), lambda i: (0, 0)),
        ],
        out_specs=pl.BlockSpec((_BLK, m), lambda i: (i, 0)),
        out_shape=jax.ShapeDtypeStruct((n, m), jnp.float32),
    )(xx, wcat, bcat)


def _final_body(pa_ref, h_ref, wrel_ref, wroot_ref, b_ref, o_ref):
    agg = pa_ref[0] + pa_ref[1]
    o = lax.dot_general(
        agg, wrel_ref[...], (((1,), (1,)), ((), ())),
        preferred_element_type=jnp.float32)
    o = o + lax.dot_general(
        h_ref[...], wroot_ref[...], (((1,), (1,)), ((), ())),
        preferred_element_type=jnp.float32)
    o = o + b_ref[...]
    mask = lax.broadcasted_iota(jnp.int32, o.shape, 1) < C
    neg = jnp.where(mask, o, -jnp.inf)
    m = jnp.max(neg, axis=1, keepdims=True)
    e = jnp.where(mask, jnp.exp(o - m), 0.0)
    ssum = jnp.sum(e, axis=1, keepdims=True)
    o_ref[...] = o - m - jnp.log(ssum)


def _final(partial2, h, wrel, wroot, b):
    return pl.pallas_call(
        _final_body,
        grid=(N // _BLK,),
        in_specs=[
            pl.BlockSpec((2, _BLK, H), lambda i: (0, i, 0)),
            pl.BlockSpec((_BLK, H), lambda i: (i, 0)),
            pl.BlockSpec((H, H), lambda i: (0, 0)),
            pl.BlockSpec((H, H), lambda i: (0, 0)),
            pl.BlockSpec((1, H), lambda i: (0, 0)),
        ],
        out_specs=pl.BlockSpec((_BLK, H), lambda i: (i, 0)),
        out_shape=jax.ShapeDtypeStruct((N, H), jnp.float32),
    )(partial2, h, wrel, wroot, b)


def kernel(x, edge_index, W1_rel, b1, W1_root, W2_rel, b2, W2_root):
    src = edge_index[0]
    dst = edge_index[1]
    zrows = jnp.zeros((_ZROWS, H), jnp.float32)

    wcat1 = jnp.concatenate([W1_rel, W1_root], axis=0)
    bcat1 = jnp.concatenate([jnp.zeros_like(b1), b1]).reshape(1, 2 * H)
    out1 = _proj(x, wcat1, bcat1)
    p1 = out1[:, :H]
    r1 = out1[:, H:]
    partial1 = _seg_partial(p1, src, dst, zrows)

    partial2, h2 = _seg_fused(partial1, r1, src, dst, zrows)

    w2rel = jnp.pad(W2_rel, ((0, H - C), (0, 0)))
    w2root = jnp.pad(W2_root, ((0, H - C), (0, 0)))
    b2p = jnp.pad(b2, (0, H - C)).reshape(1, H)
    out16 = _final(partial2, h2[0], w2rel, w2root, b2p)
    return out16[:, :C]
